# Initial kernel scaffold; baseline (speedup 1.0000x reference)
import jax
import jax.numpy as jnp
from jax.experimental import pallas as pl


def kernel(x, table):
    raise NotImplementedError("write your pallas kernel here")


# speedup vs baseline: 1.4551x; 1.4551x over previous
import functools
import math

import jax
import jax.numpy as jnp
from jax import lax
from jax.experimental import pallas as pl
from jax.experimental.pallas import tpu as pltpu
from jax.experimental.pallas import tpu_sc as plsc

_NUM_CORES = 2
_NUM_SUBCORES = 16
_LANES = 16
_NUM_WORKERS = _NUM_CORES * _NUM_SUBCORES

_CHUNK = 32
_NBUF = 3


@functools.lru_cache(maxsize=None)
def _make_lookup(vocab, d_model, batch):
  assert batch % _NUM_WORKERS == 0
  b_per_w = batch // _NUM_WORKERS
  assert b_per_w % _CHUNK == 0
  n_chunks = b_per_w // _CHUNK
  n_vecs = d_model // _LANES
  scale = math.sqrt(float(d_model))

  mesh = plsc.VectorSubcoreMesh(core_axis_name="c", subcore_axis_name="s")

  @functools.partial(
      pl.kernel,
      mesh=mesh,
      out_type=jax.ShapeDtypeStruct((batch, d_model), jnp.float32),
      scratch_types=[
          pltpu.VMEM((b_per_w,), jnp.int32),
          pltpu.VMEM((_NBUF, _CHUNK, d_model), jnp.float32),
          [pltpu.SemaphoreType.DMA for _ in range(_NBUF)],
      ],
  )
  def lookup(table_hbm, idx_hbm, out_hbm, idx_v, rows_v, sems):
    wid = lax.axis_index("s") * _NUM_CORES + lax.axis_index("c")
    base = wid * b_per_w
    pltpu.sync_copy(idx_hbm.at[pl.ds(base, b_per_w)], idx_v)

    def start_gather(c):
      return pltpu.async_copy(
          table_hbm.at[idx_v.at[pl.ds(c * _CHUNK, _CHUNK)]],
          rows_v.at[c % _NBUF],
          sems[c % _NBUF],
      )

    copies = {}
    copies[0] = start_gather(0)
    if n_chunks > 1:
      copies[1] = start_gather(1)

    for c in range(n_chunks):
      buf = c % _NBUF
      copies.pop(c).wait()
      if c + 2 < n_chunks:
        copies[c + 2] = start_gather(c + 2)

      def scale_row(r, _):
        for j in range(n_vecs):
          sl = pl.ds(j * _LANES, _LANES)
          rows_v[buf, r, sl] = rows_v[buf, r, sl] * scale
        return _

      lax.fori_loop(0, _CHUNK, scale_row, 0)
      pltpu.sync_copy(rows_v.at[buf],
                      out_hbm.at[pl.ds(base + c * _CHUNK, _CHUNK)])

  return lookup


def kernel(x, table):
  vocab, d_model = table.shape
  x_flat = x.reshape(-1).astype(jnp.int32)
  out = _make_lookup(vocab, d_model, x_flat.shape[0])(table, x_flat)
  return out.reshape(*x.shape, d_model)
